# baseline (speedup 1.0000x reference)
import jax
import jax.numpy as jnp
from jax import lax
from jax.experimental import pallas as pl
from jax.experimental.pallas import tpu as pltpu
from jax.experimental.pallas import tpu_sc as plsc

_N = 1000
_F = 26
_NCOLS = _F * _N
_NW = 32
_NSUB = 16
_IDXW = 32
_ZROWS = 32
_ZW = _ZROWS * _NCOLS


def _make_sc_kernel(b_per_w):
    mesh = plsc.VectorSubcoreMesh(core_axis_name="c", subcore_axis_name="s")
    own_w = b_per_w * _NCOLS
    zinit = _ZW // _NSUB

    def body(sh_hbm, out_hbm, idx_v, ones_v, zbuf, zsh, sem_z, sem_i):
        s = lax.axis_index("s")
        wid = s * 2 + lax.axis_index("c")
        base = wid * b_per_w
        own0 = base * _NCOLS

        pltpu.sync_copy(sh_hbm.at[pl.ds(base * _IDXW, b_per_w * _IDXW)],
                        idx_v)
        ones_v[...] = jnp.full((16,), 1.0, jnp.float32)

        zeros = jnp.zeros((16,), jnp.float32)

        @pl.loop(0, zinit // 16)
        def _(i):
            zbuf[pl.ds(i * 16, 16)] = zeros

        pltpu.sync_copy(zbuf, zsh.at[pl.ds(s * zinit, zinit)])
        plsc.subcore_barrier()

        chunks = []
        off = 0
        while off < own_w:
            chunks.append((off, min(_ZW, own_w - off)))
            off += _ZW
        for off, step in chunks:
            pltpu.async_copy(zsh.at[pl.ds(0, step)],
                             out_hbm.at[pl.ds(own0 + off, step)], sem_z)
        for off, step in chunks:
            pltpu.make_async_copy(zsh.at[pl.ds(0, step)],
                                  out_hbm.at[pl.ds(own0 + off, step)],
                                  sem_z).wait()

        @pl.loop(0, b_per_w)
        def _(r):
            i0 = idx_v[pl.ds(r * _IDXW, 16)]
            i1 = idx_v[pl.ds(r * _IDXW + 16, 16)]
            pltpu.async_copy(ones_v, out_hbm.at[i0], sem_i)
            pltpu.async_copy(ones_v, out_hbm.at[i1], sem_i)

        @pl.loop(0, b_per_w)
        def _(r):
            i0 = idx_v[pl.ds(r * _IDXW, 16)]
            pltpu.make_async_copy(ones_v, out_hbm.at[i0], sem_i).wait()
            pltpu.make_async_copy(ones_v, out_hbm.at[i0], sem_i).wait()

    return pl.kernel(
        body,
        out_type=jax.ShapeDtypeStruct((b_per_w * _NW * _NCOLS,), jnp.float32),
        mesh=mesh,
        scratch_types=[
            pltpu.VMEM((b_per_w * _IDXW,), jnp.int32),
            pltpu.VMEM((16,), jnp.float32),
            pltpu.VMEM((zinit,), jnp.float32),
            pltpu.VMEM_SHARED((_ZW,), jnp.float32),
            pltpu.SemaphoreType.DMA,
            pltpu.SemaphoreType.DMA,
        ],
        compiler_params=pltpu.CompilerParams(needs_layout_passes=False),
    )


def kernel(x):
    b, f = x.shape
    assert f == _F

    pos = (x + (_N * jnp.arange(f, dtype=x.dtype))[None, :]
           + (_NCOLS * jnp.arange(b, dtype=x.dtype))[:, None])
    sh = jnp.concatenate(
        [pos, jnp.broadcast_to(pos[:, :1], (b, _IDXW - f))], axis=1)

    bp = -(-b // _NW) * _NW
    if bp != b:
        padpos = (_NCOLS * jnp.arange(b, bp, dtype=x.dtype))[:, None]
        sh = jnp.concatenate(
            [sh, jnp.broadcast_to(padpos, (bp - b, _IDXW))], axis=0)

    out = _make_sc_kernel(bp // _NW)(sh.reshape(-1))
    return out.reshape(bp, _NCOLS)[:b]


# speedup vs baseline: 2.6083x; 2.6083x over previous
import jax
import jax.numpy as jnp
from jax import lax
from jax.experimental import pallas as pl
from jax.experimental.pallas import tpu as pltpu
from jax.experimental.pallas import tpu_sc as plsc

_N = 1000
_F = 26
_NCOLS = _F * _N
_NPAD = 26112
_NW = 32
_BAND = 8
_LPB = _BAND * _F
_CW = 12800
_TW = 512


def _make_sc_kernel(b_per_w):
    assert b_per_w % _BAND == 0
    nbands = b_per_w // _BAND
    mesh = plsc.VectorSubcoreMesh(core_axis_name="c", subcore_axis_name="s")

    chunks = [(0, _CW), (_CW, _CW), (2 * _CW, _TW)]

    def body(cols_hbm, out_hbm, rowv, colv, buf, sem):
        wid = lax.axis_index("s") * 2 + lax.axis_index("c")
        base = wid * b_per_w
        pltpu.sync_copy(cols_hbm.at[pl.ds(0, _LPB)], rowv)
        pltpu.sync_copy(
            cols_hbm.at[pl.ds(_LPB + base * _F, b_per_w * _F)], colv)

        ones = jnp.full((16,), 1.0, jnp.float32)
        zeros = jnp.zeros((16,), jnp.float32)

        for r in range(_BAND):
            @pl.loop(0, _CW // 16)
            def _(i, r=r):
                buf[r, pl.ds(i * 16, 16)] = zeros

        def scat(bnd, c0, w, val):
            for g in range(_LPB // 16):
                cv = colv[pl.ds(bnd * _LPB + g * 16, 16)]
                rv = rowv[pl.ds(g * 16, 16)]
                m = (cv >= c0) & (cv < c0 + w)
                plsc.store_scatter(buf, [rv, cv - c0], val, mask=m)

        @pl.loop(0, nbands)
        def _(bnd):
            r0 = base + bnd * _BAND
            for c0, w in chunks:
                scat(bnd, c0, w, ones)
                pltpu.async_copy(
                    buf.at[pl.ds(0, _BAND), pl.ds(0, w)],
                    out_hbm.at[pl.ds(r0, _BAND), pl.ds(c0, w)], sem)
                pltpu.make_async_copy(
                    buf.at[pl.ds(0, _BAND), pl.ds(0, w)],
                    out_hbm.at[pl.ds(r0, _BAND), pl.ds(c0, w)], sem).wait()
                scat(bnd, c0, w, zeros)

    return pl.kernel(
        body,
        out_type=jax.ShapeDtypeStruct((b_per_w * _NW, _NPAD), jnp.float32),
        mesh=mesh,
        scratch_types=[
            pltpu.VMEM((_LPB,), jnp.int32),
            pltpu.VMEM((b_per_w * _F,), jnp.int32),
            pltpu.VMEM((_BAND, _CW), jnp.float32),
            pltpu.SemaphoreType.DMA,
        ],
        compiler_params=pltpu.CompilerParams(
            needs_layout_passes=False, use_tc_tiling_on_sc=True),
    )


def kernel(x):
    b, f = x.shape
    assert f == _F

    cols = x + (_N * jnp.arange(f, dtype=x.dtype))[None, :]

    bp = -(-b // (_NW * _BAND)) * (_NW * _BAND)
    if bp != b:
        cols = jnp.pad(cols, ((0, bp - b), (0, 0)),
                       constant_values=2 * _NPAD)

    header = (jnp.arange(_LPB, dtype=jnp.int32) // _F) % _BAND
    table = jnp.concatenate([header, cols.reshape(-1)])
    out = _make_sc_kernel(bp // _NW)(table)
    return out[:b, :_NCOLS]
